# baseline (speedup 1.0000x reference)
import functools
import math

import jax
import jax.numpy as jnp
import numpy as np
from jax import lax
from jax.experimental import pallas as pl
from jax.experimental.pallas import tpu as pltpu
from jax.experimental.pallas import tpu_sc as plsc

_NB = 64
_NSEG = _NB - 1
_NS = 20
_NOUT = _NSEG * _NS
_OC = 2 * _NOUT
_IC = 2 * _NB

_Z1 = -2.0 + math.sqrt(3.0)
_NC, _NSUB, _L = 2, 16, 16
_NW = _NC * _NSUB
_CB = _L


def _sc_block_compute(in_v, out_v, rt_v, qt_v, q_v, lanes):
    z1 = np.float32(_Z1)
    six_z1 = np.float32(6.0 * _Z1)
    n = _NB
    rowb_in = lanes * _IC
    rowb_out = lanes * _OC

    for c in range(2):
        somme = jnp.zeros((_L,), jnp.float32)
        pw = 1.0
        for i in range(n):
            v = plsc.load_gather(in_v, [rowb_in + (2 * i + c)])
            rt_v[pl.ds(_L * i, _L)] = v
            somme = somme + np.float32(pw) * v
            pw *= _Z1
        qt = somme * np.float32(1.0 / (1.0 - _Z1 ** n))
        qt_v[pl.ds(0, _L)] = qt
        somme2 = qt
        pw = _Z1
        for i in range(1, n):
            qt = z1 * qt + rt_v[pl.ds(_L * i, _L)]
            qt_v[pl.ds(_L * i, _L)] = qt
            somme2 = somme2 + np.float32(pw) * qt
            pw *= _Z1
        q0 = np.float32(-(6.0 * _Z1 / (1.0 - _Z1 ** n))) * somme2
        qtn = z1 * q0 - six_z1 * qt
        q_v[pl.ds(0, _L)] = q0
        carry = qtn
        for i in range(n - 2, 0, -1):
            carry = z1 * carry - six_z1 * qt_v[pl.ds(_L * i, _L)]
            q_v[pl.ds(_L * i, _L)] = carry

        def seg_body(seg, _):
            r1 = lax.rem(seg + 1, _NSEG)
            r2 = lax.rem(seg + 2, _NSEG)
            r3 = lax.rem(seg + 3, _NSEG)
            Q0 = plsc.load_gather(q_v, [_L * seg + lanes])
            Q1 = plsc.load_gather(q_v, [_L * r1 + lanes])
            Q2 = plsc.load_gather(q_v, [_L * r2 + lanes])
            Q3 = plsc.load_gather(q_v, [_L * r3 + lanes])
            c3 = (Q3 - Q0) * np.float32(1.0 / 6.0) + (Q1 - Q2) * np.float32(0.5)
            c2 = (Q0 + Q2) * np.float32(0.5) - Q1
            c1 = (Q2 - Q0) * np.float32(0.5)
            c0 = (Q0 + Q2) * np.float32(1.0 / 6.0) + Q1 * np.float32(2.0 / 3.0)
            idx0 = rowb_out + (seg * (2 * _NS) + c)
            for t in range(_NS):
                s = np.float32(t / (_NS - 1.0))
                val = ((c3 * s + c2) * s + c1) * s + c0
                plsc.store_scatter(out_v, [idx0 + 2 * t], val)
            return 0

        lax.fori_loop(0, _NSEG, seg_body, 0)


def _sc_body(nblk, in_hbm, out_hbm, in_v, out_v, rt_v, qt_v, q_v):
    wid = lax.axis_index("s") * _NC + lax.axis_index("c")
    lanes = lax.iota(jnp.int32, _L)

    def block(blk, _):
        base = (wid * nblk + blk) * _CB
        pltpu.sync_copy(in_hbm.at[pl.ds(base * _IC, _CB * _IC)], in_v)
        _sc_block_compute(in_v, out_v, rt_v, qt_v, q_v, lanes)
        pltpu.sync_copy(out_v, out_hbm.at[pl.ds(base * _OC, _CB * _OC)])
        return 0

    lax.fori_loop(0, nblk, block, 0)


def _sc_curve(x_flat, B):
    assert B % (_NW * _CB) == 0
    nblk = B // (_NW * _CB)
    mesh = plsc.VectorSubcoreMesh(core_axis_name="c", subcore_axis_name="s",
                                  num_cores=_NC, num_subcores=_NSUB)
    fn = pl.kernel(
        functools.partial(_sc_body, nblk),
        out_type=jax.ShapeDtypeStruct((B * _OC,), jnp.float32),
        mesh=mesh,
        scratch_types=[
            pltpu.VMEM((_CB * _IC,), jnp.float32),
            pltpu.VMEM((_CB * _OC,), jnp.float32),
            pltpu.VMEM((_NB * _L,), jnp.float32),
            pltpu.VMEM((_NB * _L,), jnp.float32),
            pltpu.VMEM((_NB * _L,), jnp.float32),
        ],
        compiler_params=pltpu.CompilerParams(needs_layout_passes=False),
    )
    return fn(x_flat)


@functools.lru_cache(maxsize=None)
def _spline_matrix() -> np.ndarray:
    n = _NB
    z1 = -2.0 + np.sqrt(3.0)
    R = np.eye(n, dtype=np.float64)
    powers = z1 ** np.arange(n, dtype=np.float64)

    qt0 = (powers @ R) / (1.0 - z1 ** n)
    QT = np.zeros((n, n), dtype=np.float64)
    QT[0] = qt0
    for i in range(1, n):
        QT[i] = z1 * QT[i - 1] + R[i]
    q0 = -(6.0 * z1 / (1.0 - z1 ** n)) * (powers @ QT)
    qtn = z1 * q0 - 6.0 * z1 * QT[n - 1]
    Q = np.zeros((n, n), dtype=np.float64)
    Q[0] = q0
    Q[n - 1] = qtn
    carry = qtn
    for i in range(n - 2, 0, -1):
        carry = z1 * carry - 6.0 * z1 * QT[i]
        Q[i] = carry

    s = np.linspace(0.0, 1.0, _NS)
    idx = (np.arange(_NSEG)[:, None] + np.arange(4)[None, :]) % _NSEG
    Qs = Q[idx]
    Q0, Q1, Q2, Q3 = Qs[:, 0], Qs[:, 1], Qs[:, 2], Qs[:, 3]
    c3 = -Q0 / 6.0 + Q1 / 2.0 - Q2 / 2.0 + Q3 / 6.0
    c2 = Q0 / 2.0 - Q1 + Q2 / 2.0
    c1 = -Q0 / 2.0 + Q2 / 2.0
    c0 = Q0 / 6.0 + 2.0 * Q1 / 3.0 + Q2 / 6.0
    curve = (c3[:, None] * (s ** 3)[None, :, None]
             + c2[:, None] * (s ** 2)[None, :, None]
             + c1[:, None] * s[None, :, None]
             + c0[:, None])
    return curve.reshape(_NOUT, n).T


@functools.lru_cache(maxsize=None)
def _interleaved_operator() -> np.ndarray:
    M = _spline_matrix()
    W = np.zeros((2 * _NB, 2 * _NOUT), dtype=np.float64)
    W[0::2, 0::2] = M
    W[1::2, 1::2] = M
    return W.astype(np.float32)


def _matmul_body(w_ref, x_ref, o_ref):
    xb = x_ref[:, 0, 0, :].astype(jnp.bfloat16)
    res = jnp.dot(w_ref[...], xb, preferred_element_type=jnp.float32)
    o_ref[:, 0, 0, :] = res


def _tc_curve_native(x3, nrow):
    Mt = jnp.asarray(np.ascontiguousarray(_spline_matrix().T))
    Mt = Mt.astype(jnp.bfloat16)
    x4 = x3.reshape(_NB, nrow, 1, 128)
    return pl.pallas_call(
        _matmul_body,
        grid=(nrow,),
        in_specs=[
            pl.BlockSpec((_NOUT, _NB), lambda i: (0, 0)),
            pl.BlockSpec((_NB, 1, 1, 128), lambda i: (0, i, 0, 0)),
        ],
        out_specs=pl.BlockSpec((_NOUT, 1, 1, 128), lambda i: (0, i, 0, 0)),
        out_shape=jax.ShapeDtypeStruct((_NOUT, nrow, 1, 128), jnp.float32),
    )(Mt, x4).reshape(_NOUT, nrow, 128)


def kernel(inputs):
    B = inputs.shape[0]
    nbt = B // 128
    x3 = (inputs.transpose(1, 2, 0).reshape(_NB, 2, nbt, 128)
          .transpose(0, 2, 1, 3).reshape(_NB, nbt * 2, 128))
    out = _tc_curve_native(x3, nbt * 2)
    out = (out.reshape(_NOUT, nbt, 2, 128).transpose(1, 3, 0, 2)
           .reshape(B, _NOUT, 1, 2))
    return out


# speedup vs baseline: 3.7418x; 3.7418x over previous
import functools
import math

import jax
import jax.numpy as jnp
import numpy as np
from jax import lax
from jax.experimental import pallas as pl
from jax.experimental.pallas import tpu as pltpu
from jax.experimental.pallas import tpu_sc as plsc

_NB = 64
_NSEG = _NB - 1
_NS = 20
_NOUT = _NSEG * _NS
_OC = 2 * _NOUT
_IC = 2 * _NB

_Z1 = -2.0 + math.sqrt(3.0)
_NC, _NSUB, _L = 2, 16, 16
_NW = _NC * _NSUB
_CB = _L


def _sc_block_compute(in_v, out_v, rt_v, qt_v, q_v, lanes):
    z1 = np.float32(_Z1)
    six_z1 = np.float32(6.0 * _Z1)
    n = _NB
    rowb_in = lanes * _IC
    rowb_out = lanes * _OC

    for c in range(2):
        somme = jnp.zeros((_L,), jnp.float32)
        pw = 1.0
        for i in range(n):
            v = plsc.load_gather(in_v, [rowb_in + (2 * i + c)])
            rt_v[pl.ds(_L * i, _L)] = v
            somme = somme + np.float32(pw) * v
            pw *= _Z1
        qt = somme * np.float32(1.0 / (1.0 - _Z1 ** n))
        qt_v[pl.ds(0, _L)] = qt
        somme2 = qt
        pw = _Z1
        for i in range(1, n):
            qt = z1 * qt + rt_v[pl.ds(_L * i, _L)]
            qt_v[pl.ds(_L * i, _L)] = qt
            somme2 = somme2 + np.float32(pw) * qt
            pw *= _Z1
        q0 = np.float32(-(6.0 * _Z1 / (1.0 - _Z1 ** n))) * somme2
        qtn = z1 * q0 - six_z1 * qt
        q_v[pl.ds(0, _L)] = q0
        carry = qtn
        for i in range(n - 2, 0, -1):
            carry = z1 * carry - six_z1 * qt_v[pl.ds(_L * i, _L)]
            q_v[pl.ds(_L * i, _L)] = carry

        def seg_body(seg, _):
            r1 = lax.rem(seg + 1, _NSEG)
            r2 = lax.rem(seg + 2, _NSEG)
            r3 = lax.rem(seg + 3, _NSEG)
            Q0 = plsc.load_gather(q_v, [_L * seg + lanes])
            Q1 = plsc.load_gather(q_v, [_L * r1 + lanes])
            Q2 = plsc.load_gather(q_v, [_L * r2 + lanes])
            Q3 = plsc.load_gather(q_v, [_L * r3 + lanes])
            c3 = (Q3 - Q0) * np.float32(1.0 / 6.0) + (Q1 - Q2) * np.float32(0.5)
            c2 = (Q0 + Q2) * np.float32(0.5) - Q1
            c1 = (Q2 - Q0) * np.float32(0.5)
            c0 = (Q0 + Q2) * np.float32(1.0 / 6.0) + Q1 * np.float32(2.0 / 3.0)
            idx0 = rowb_out + (seg * (2 * _NS) + c)
            for t in range(_NS):
                s = np.float32(t / (_NS - 1.0))
                val = ((c3 * s + c2) * s + c1) * s + c0
                plsc.store_scatter(out_v, [idx0 + 2 * t], val)
            return 0

        lax.fori_loop(0, _NSEG, seg_body, 0)


def _sc_body(nblk, in_hbm, out_hbm, in_v, out_v, rt_v, qt_v, q_v):
    wid = lax.axis_index("s") * _NC + lax.axis_index("c")
    lanes = lax.iota(jnp.int32, _L)

    def block(blk, _):
        base = (wid * nblk + blk) * _CB
        pltpu.sync_copy(in_hbm.at[pl.ds(base * _IC, _CB * _IC)], in_v)
        _sc_block_compute(in_v, out_v, rt_v, qt_v, q_v, lanes)
        pltpu.sync_copy(out_v, out_hbm.at[pl.ds(base * _OC, _CB * _OC)])
        return 0

    lax.fori_loop(0, nblk, block, 0)


def _sc_curve(x_flat, B):
    assert B % (_NW * _CB) == 0
    nblk = B // (_NW * _CB)
    mesh = plsc.VectorSubcoreMesh(core_axis_name="c", subcore_axis_name="s",
                                  num_cores=_NC, num_subcores=_NSUB)
    fn = pl.kernel(
        functools.partial(_sc_body, nblk),
        out_type=jax.ShapeDtypeStruct((B * _OC,), jnp.float32),
        mesh=mesh,
        scratch_types=[
            pltpu.VMEM((_CB * _IC,), jnp.float32),
            pltpu.VMEM((_CB * _OC,), jnp.float32),
            pltpu.VMEM((_NB * _L,), jnp.float32),
            pltpu.VMEM((_NB * _L,), jnp.float32),
            pltpu.VMEM((_NB * _L,), jnp.float32),
        ],
        compiler_params=pltpu.CompilerParams(needs_layout_passes=False),
    )
    return fn(x_flat)


@functools.lru_cache(maxsize=None)
def _spline_matrix() -> np.ndarray:
    n = _NB
    z1 = -2.0 + np.sqrt(3.0)
    R = np.eye(n, dtype=np.float64)
    powers = z1 ** np.arange(n, dtype=np.float64)

    qt0 = (powers @ R) / (1.0 - z1 ** n)
    QT = np.zeros((n, n), dtype=np.float64)
    QT[0] = qt0
    for i in range(1, n):
        QT[i] = z1 * QT[i - 1] + R[i]
    q0 = -(6.0 * z1 / (1.0 - z1 ** n)) * (powers @ QT)
    qtn = z1 * q0 - 6.0 * z1 * QT[n - 1]
    Q = np.zeros((n, n), dtype=np.float64)
    Q[0] = q0
    Q[n - 1] = qtn
    carry = qtn
    for i in range(n - 2, 0, -1):
        carry = z1 * carry - 6.0 * z1 * QT[i]
        Q[i] = carry

    s = np.linspace(0.0, 1.0, _NS)
    idx = (np.arange(_NSEG)[:, None] + np.arange(4)[None, :]) % _NSEG
    Qs = Q[idx]
    Q0, Q1, Q2, Q3 = Qs[:, 0], Qs[:, 1], Qs[:, 2], Qs[:, 3]
    c3 = -Q0 / 6.0 + Q1 / 2.0 - Q2 / 2.0 + Q3 / 6.0
    c2 = Q0 / 2.0 - Q1 + Q2 / 2.0
    c1 = -Q0 / 2.0 + Q2 / 2.0
    c0 = Q0 / 6.0 + 2.0 * Q1 / 3.0 + Q2 / 6.0
    curve = (c3[:, None] * (s ** 3)[None, :, None]
             + c2[:, None] * (s ** 2)[None, :, None]
             + c1[:, None] * s[None, :, None]
             + c0[:, None])
    return curve.reshape(_NOUT, n).T


def _matmul_body(w_ref, x_ref, o_ref):
    xb = x_ref[...].reshape(_NB, -1).astype(jnp.bfloat16)
    res = jnp.dot(w_ref[...], xb, preferred_element_type=jnp.float32)
    o_ref[...] = res.reshape(o_ref.shape)


def _tc_curve_native(x3, nrow):
    Mt = jnp.asarray(np.ascontiguousarray(_spline_matrix().T))
    Mt = Mt.astype(jnp.bfloat16)
    bn = 16
    return pl.pallas_call(
        _matmul_body,
        grid=(nrow // bn,),
        in_specs=[
            pl.BlockSpec((_NOUT, _NB), lambda i: (0, 0)),
            pl.BlockSpec((_NB, bn, 128), lambda i: (0, i, 0)),
        ],
        out_specs=pl.BlockSpec((_NOUT, bn, 128), lambda i: (0, i, 0)),
        out_shape=jax.ShapeDtypeStruct((_NOUT, nrow, 128), jnp.float32),
    )(Mt, x3)


def kernel(inputs):
    B = inputs.shape[0]
    nbt = B // 128
    x3 = (inputs.transpose(1, 2, 0).reshape(_NB, 2, nbt, 128)
          .transpose(0, 2, 1, 3).reshape(_NB, nbt * 2, 128))
    out = _tc_curve_native(x3, nbt * 2)
    out = (out.reshape(_NOUT, nbt, 2, 128).transpose(1, 3, 0, 2)
           .reshape(B, _NOUT, 1, 2))
    return out
